# baseline (speedup 1.0000x reference)
import jax
import jax.numpy as jnp
import numpy as np
from jax.experimental import pallas as pl

B, N, H = 2, 128, 4
SP, TD, DE = 16, 8, 16
CUT = 5.0
DK = 8
DV0, DV1 = 8, 2

TI = 16
S = TI * N

RSTEP = np.float32(CUT / (DE - 1))
SQRT26 = np.float32(np.sqrt(26.0))
SQRT78 = np.float32(np.sqrt(3.0) * np.sqrt(26.0))
ASCL = np.float32(1.0 / (np.sqrt(64.0) * np.sqrt(float(DK))))


def _peratom_kernel(c24_ref, wcat_ref, out_ref):
    out_ref[...] = jnp.dot(c24_ref[...], wcat_ref[...],
                           preferred_element_type=jnp.float32,
                           precision=jax.lax.Precision.HIGHEST)


def _ii(shape, dim):
    return jax.lax.broadcasted_iota(jnp.int32, shape, dim)


def _fdiv(x, n):
    return (x.astype(jnp.float32) * np.float32(1.0 / n)).astype(jnp.int32)


def _mm(a, b):
    return jnp.dot(a, b, preferred_element_type=jnp.float32,
                   precision=jax.lax.Precision.HIGHEST)


def _mmx(a, b):
    return jnp.dot(a, b, preferred_element_type=jnp.float32,
                   precision=jax.lax.Precision.HIGHEST)


def _edge_kernel(coord_ref, veloc_ref, qw_ref,
                 ak_ref, akb_ref, av0_ref, avb0_ref, ave_ref, avbe_ref,
                 tk_ref, tv_ref,
                 mk1_ref, mkb1_ref, mv1_ref, mvb1_ref,
                 bk34_ref, bk34b_ref, bv34_ref, bv34b_ref,
                 u34_ref, u34b_ref, wout0_ref, w1o_ref,
                 out0_ref, out1_ref, attn_ref):
    g = pl.program_id(1)
    f32 = jnp.float32

    io0 = _ii((S, N), 0)
    io1 = _ii((S, N), 1)
    jmod = io0 - _fdiv(io0, N) * N
    eyeT = (jmod == io1).astype(f32)
    seg = (_fdiv(_ii((TI, S), 1), N) == _ii((TI, S), 0)).astype(f32)
    segT = (_fdiv(_ii((S, TI), 0), N) == _ii((S, TI), 1)).astype(f32)
    selm = (_fdiv(_ii((32, H), 0), DK) == _ii((32, H), 1)).astype(f32)
    rep8 = (_fdiv(_ii((H, 32), 1), DK) == _ii((H, 32), 0)).astype(f32)
    rep6 = (_fdiv(_ii((H, 24), 1), 6) == _ii((H, 24), 0)).astype(f32)
    r8 = (_fdiv(_ii((DV0, 24), 1), 3) == _ii((DV0, 24), 0)).astype(f32)
    x24 = _ii((3, 24), 1)
    t3 = ((x24 - _fdiv(x24, 3) * 3) == _ii((3, 24), 0)).astype(f32)

    cb = coord_ref[0]
    vb = veloc_ref[0]
    qwb = qw_ref[0]
    cis = coord_ref[0, pl.ds(g * TI, TI), :]
    vis = veloc_ref[0, pl.ds(g * TI, TI), :]

    ce = _mmx(eyeT, cb) - _mmx(segT, cis)
    ve = _mmx(eyeT, vb) - _mmx(segT, vis)
    qw_t = _mm(eyeT, qwb)

    sumsq = jnp.sum(ce * ce, axis=1, keepdims=True)
    d = jnp.sqrt(sumsq + np.float32(1e-12))
    dot_ve = jnp.sum(ve * ce, axis=1, keepdims=True)
    inv2s3d = np.float32(-0.5 / np.sqrt(3.0)) / d
    s1a = sumsq * inv2s3d
    s1b = dot_ve * inv2s3d

    kvals = _ii((1, DE), 1).astype(f32) * RSTEP
    diff = (d - kvals) * np.float32(1.0 / RSTEP)
    rbf = jnp.exp(-diff * diff) * np.float32(1.0 / 1.12)

    tkx = _mm(segT, tk_ref[0, :, 0, :])
    tvx = _mm(segT, tv_ref[0, :, 0, :])
    akbx = _mm(segT, akb_ref[0, :, 0, :])
    avb0x = _mm(segT, avb0_ref[0, :, 0, :])
    avbex = _mm(segT, avbe_ref[0, :, 0, :])

    hk = jax.nn.silu(_mm(rbf, mk1_ref[...]) + tkx + mkb1_ref[...])
    hv = jax.nn.silu(_mm(rbf, mv1_ref[...]) + tvx + mvb1_ref[...])

    g34 = _mm(hk, bk34_ref[...]) + bk34b_ref[...]
    gi = jnp.concatenate(
        [_mm(hk[t * N:(t + 1) * N], ak_ref[0, t]) for t in range(TI)], axis=0)
    keyf = (gi + akbx + s1a * g34[:, :32] + s1b * g34[:, 32:]) * (1.0 / SQRT26)

    a = _mm(qw_t * keyf, selm) * ASCL
    amax = jnp.max(a, axis=0, keepdims=True)
    ex = jnp.exp(a - amax)
    denom = _mm(segT, _mm(seg, ex))
    attn = ex / denom

    gv34 = _mm(hv, bv34_ref[...]) + bv34b_ref[...]
    gvi = jnp.concatenate(
        [_mm(hv[t * N:(t + 1) * N], av0_ref[0, t]) for t in range(TI)], axis=0)
    val0 = (gvi + avb0x + s1a * gv34[:, :32] + s1b * gv34[:, 32:]) * (
        1.0 / SQRT26)

    e1 = jnp.concatenate(
        [_mm(hv[t * N:(t + 1) * N], ave_ref[0, t]) for t in range(TI)], axis=0)
    e1 = e1 + avbex
    gu = _mm(hv, u34_ref[...]) + u34b_ref[...]
    coefc = e1 / d - np.float32(0.5) * gu[:, :DV0]
    u4h = np.float32(0.5) * gu[:, DV0:]
    val1 = (_mm(coefc, r8) * _mm(ce, t3)
            - _mm(u4h, r8) * _mm(ve, t3)) * (1.0 / SQRT78)

    at32 = _mm(attn, rep8)
    at24 = _mm(attn, rep6)
    x0 = _mm(seg, at32 * val0)
    x1 = _mm(seg, at24 * val1)

    out0 = _mm(x0, wout0_ref[...]) * np.float32(1.0 / np.sqrt(32.0))
    out0_ref[0] = out0.reshape(TI, 1, 24)
    out1_ref[0] = _mm(x1, w1o_ref[...]).reshape(TI, 1, 6)

    dmat = _mmx(segT, _mmx(seg, d * eyeT))
    less = ((dmat < d) | ((dmat == d) & (io1 < jmod))).astype(f32)
    rank = jnp.sum(less, axis=1, keepdims=True)
    rank_rows = _mm(seg, rank * eyeT)
    r_io = _ii((N, N), 0).astype(f32)
    for t in range(TI):
        perm = (rank_rows[t:t + 1, :] == r_io).astype(f32)
        attn_ref[0, t] = _mm(perm, attn[t * N:(t + 1) * N])


@jax.jit
def kernel(sp, coord, veloc, t, Wq, mkW1, mkb1, mkW2, mkb2,
           mvW1, mvb1, mvW2, mvb2, w_dot, Wout0e, Wout1o):
    f32 = jnp.float32

    wk_atom = mkW2[:, :768].reshape(64, 24, 32).transpose(1, 0, 2).reshape(24, 2048)
    bk_atom = mkb2[:768].reshape(24, 32)
    bk34 = mkW2[:, 768:832]
    bk34b = mkb2[768:832].reshape(1, 64)
    wv_atom0 = mvW2[:, :768].reshape(64, 24, 32).transpose(1, 0, 2).reshape(24, 2048)
    bv_atom0 = mvb2[:768].reshape(24, 32)
    bv34 = mvW2[:, 768:832]
    bv34b = mvb2[768:832].reshape(1, 64)
    wv_atome = mvW2[:, 832:1024].reshape(64, 24, 8).transpose(1, 0, 2).reshape(24, 512)
    bv_atome = mvb2[832:1024].reshape(24, 8)
    u34 = mvW2[:, 1024:1040]
    u34b = mvb2[1024:1040].reshape(1, 16)
    wq_fold = (Wq.reshape(24, H, DK) @ w_dot).reshape(24, 32) * f32(
        1.0 / np.sqrt(24.0))
    w1o_exp = jnp.einsum('ko,xy->kxoy', Wout1o,
                         jnp.eye(3, dtype=f32)).reshape(24, 6) * f32(
                             1.0 / np.sqrt(8.0))
    zpad = jnp.zeros((SP, 64), f32)
    tk_w = jnp.concatenate([zpad, mkW1[SP:, :]], 0)
    tv_w = jnp.concatenate([zpad, mvW1[SP:, :]], 0)
    wcat = jnp.concatenate([
        wk_atom, bk_atom, wv_atom0, bv_atom0, wv_atome, bv_atome,
        wq_fold, tk_w, tv_w], axis=1)

    c24 = jnp.concatenate([sp, t], -1).reshape(B * N, 24)

    pa = pl.pallas_call(
        _peratom_kernel,
        out_shape=jax.ShapeDtypeStruct((B * N, 4840), f32),
    )(c24, wcat)

    pa = pa.reshape(B, N, 4840)
    a_k = pa[..., 0:2048].reshape(B, N, 64, 32)
    akb = pa[..., 2048:2080].reshape(B, N, 1, 32)
    a_v0 = pa[..., 2080:4128].reshape(B, N, 64, 32)
    avb0 = pa[..., 4128:4160].reshape(B, N, 1, 32)
    a_ve = pa[..., 4160:4672].reshape(B, N, 64, 8)
    avbe = pa[..., 4672:4680].reshape(B, N, 1, 8)
    qw = pa[..., 4680:4712]
    tk = pa[..., 4712:4776].reshape(B, N, 1, 64)
    tv = pa[..., 4776:4840].reshape(B, N, 1, 64)

    full2 = lambda arr: pl.BlockSpec(arr.shape, lambda b, i: (0, 0))
    perb = lambda shp: pl.BlockSpec((1,) + shp[1:], lambda b, i: (b, 0, 0))
    peri = lambda shp: pl.BlockSpec((1, TI) + shp[2:],
                                    lambda b, i: (b, i, 0, 0))

    mkb1r = mkb1.reshape(1, 64)
    mvb1r = mvb1.reshape(1, 64)

    out0, out1, attn_s = pl.pallas_call(
        _edge_kernel,
        grid=(B, N // TI),
        in_specs=[
            perb(coord.shape), perb(veloc.shape), perb(qw.shape),
            peri(a_k.shape), peri(akb.shape), peri(a_v0.shape),
            peri(avb0.shape), peri(a_ve.shape), peri(avbe.shape),
            peri(tk.shape), peri(tv.shape),
            full2(mkW1[:SP]), full2(mkb1r), full2(mvW1[:SP]), full2(mvb1r),
            full2(bk34), full2(bk34b), full2(bv34), full2(bv34b),
            full2(u34), full2(u34b), full2(Wout0e), full2(w1o_exp),
        ],
        out_specs=[
            pl.BlockSpec((1, TI, 1, 24), lambda b, i: (b, i, 0, 0)),
            pl.BlockSpec((1, TI, 1, 6), lambda b, i: (b, i, 0, 0)),
            pl.BlockSpec((1, TI, N, H), lambda b, i: (b, i, 0, 0)),
        ],
        out_shape=[
            jax.ShapeDtypeStruct((B, N, 1, 24), f32),
            jax.ShapeDtypeStruct((B, N, 1, 6), f32),
            jax.ShapeDtypeStruct((B, N, N, H), f32),
        ],
    )(coord, veloc, qw, a_k, akb, a_v0, avb0, a_ve, avbe, tk, tv,
      mkW1[:SP], mkb1r, mvW1[:SP], mvb1r, bk34, bk34b, bv34, bv34b,
      u34, u34b, Wout0e, w1o_exp)

    out0 = out0.reshape(B, N, 24)
    out1 = out1.reshape(B, N, 6)
    sp_o = out0[..., :SP]
    t_o = out0[..., SP:]
    coord_o = out1[..., :3]
    veloc_o = out1[..., 3:]
    attn = attn_s.transpose(0, 3, 1, 2)[..., None]
    return (sp_o, coord_o, veloc_o, t_o, attn)


# speedup vs baseline: 2.2687x; 2.2687x over previous
import jax
import jax.numpy as jnp
import numpy as np
from jax.experimental import pallas as pl

B, N, H = 2, 128, 4
SP, TD, DE = 16, 8, 16
CUT = 5.0
DK = 8
DV0, DV1 = 8, 2

TI = 16
S = TI * N

RSTEP = np.float32(CUT / (DE - 1))
SQRT26 = np.float32(np.sqrt(26.0))
SQRT78 = np.float32(np.sqrt(3.0) * np.sqrt(26.0))
ASCL = np.float32(1.0 / (np.sqrt(64.0) * np.sqrt(float(DK))))


def _peratom_kernel(c24_ref, wcat_ref, out_ref):
    out_ref[...] = jnp.dot(c24_ref[...], wcat_ref[...],
                           preferred_element_type=jnp.float32)


def _ii(shape, dim):
    return jax.lax.broadcasted_iota(jnp.int32, shape, dim)


def _fdiv(x, n):
    return (x.astype(jnp.float32) * np.float32(1.0 / n)).astype(jnp.int32)


def _mm(a, b):
    return jnp.dot(a, b, preferred_element_type=jnp.float32)


def _mmx(a, b):
    return jnp.dot(a, b, preferred_element_type=jnp.float32,
                   precision=jax.lax.Precision.HIGHEST)


def _edge_kernel(coord_ref, veloc_ref, qw_ref,
                 ak_ref, akb_ref, av0_ref, avb0_ref, ave_ref, avbe_ref,
                 tk_ref, tv_ref,
                 mk1_ref, mkb1_ref, mv1_ref, mvb1_ref,
                 bk34_ref, bk34b_ref, bv34_ref, bv34b_ref,
                 u34_ref, u34b_ref, wout0_ref, w1o_ref,
                 out0_ref, out1_ref, attn_ref):
    g = pl.program_id(1)
    f32 = jnp.float32

    io0 = _ii((S, N), 0)
    io1 = _ii((S, N), 1)
    jmod = io0 - _fdiv(io0, N) * N
    eyeT = (jmod == io1).astype(f32)
    seg = (_fdiv(_ii((TI, S), 1), N) == _ii((TI, S), 0)).astype(f32)
    segT = (_fdiv(_ii((S, TI), 0), N) == _ii((S, TI), 1)).astype(f32)
    selm = (_fdiv(_ii((32, H), 0), DK) == _ii((32, H), 1)).astype(f32)
    rep8 = (_fdiv(_ii((H, 32), 1), DK) == _ii((H, 32), 0)).astype(f32)
    rep6 = (_fdiv(_ii((H, 24), 1), 6) == _ii((H, 24), 0)).astype(f32)
    r8 = (_fdiv(_ii((DV0, 24), 1), 3) == _ii((DV0, 24), 0)).astype(f32)
    x24 = _ii((3, 24), 1)
    t3 = ((x24 - _fdiv(x24, 3) * 3) == _ii((3, 24), 0)).astype(f32)

    cb = coord_ref[0]
    vb = veloc_ref[0]
    qwb = qw_ref[0]
    cis = coord_ref[0, pl.ds(g * TI, TI), :]
    vis = veloc_ref[0, pl.ds(g * TI, TI), :]

    ce = _mmx(eyeT, cb) - _mmx(segT, cis)
    ve = _mmx(eyeT, vb) - _mmx(segT, vis)
    qw_t = _mm(eyeT, qwb)

    sumsq = jnp.sum(ce * ce, axis=1, keepdims=True)
    d = jnp.sqrt(sumsq + np.float32(1e-12))
    dot_ve = jnp.sum(ve * ce, axis=1, keepdims=True)
    inv2s3d = np.float32(-0.5 / np.sqrt(3.0)) / d
    s1a = sumsq * inv2s3d
    s1b = dot_ve * inv2s3d

    kvals = _ii((1, DE), 1).astype(f32) * RSTEP
    diff = (d - kvals) * np.float32(1.0 / RSTEP)
    rbf = jnp.exp(-diff * diff) * np.float32(1.0 / 1.12)

    tkx = _mm(segT, tk_ref[0, :, 0, :])
    tvx = _mm(segT, tv_ref[0, :, 0, :])
    akbx = _mm(segT, akb_ref[0, :, 0, :])
    avb0x = _mm(segT, avb0_ref[0, :, 0, :])
    avbex = _mm(segT, avbe_ref[0, :, 0, :])

    hk = jax.nn.silu(_mm(rbf, mk1_ref[...]) + tkx + mkb1_ref[...])
    hv = jax.nn.silu(_mm(rbf, mv1_ref[...]) + tvx + mvb1_ref[...])

    g34 = _mm(hk, bk34_ref[...]) + bk34b_ref[...]
    gi = jnp.concatenate(
        [_mm(hk[t * N:(t + 1) * N], ak_ref[0, t]) for t in range(TI)], axis=0)
    keyf = (gi + akbx + s1a * g34[:, :32] + s1b * g34[:, 32:]) * (1.0 / SQRT26)

    a = _mm(qw_t * keyf, selm) * ASCL
    amax = jnp.max(a, axis=0, keepdims=True)
    ex = jnp.exp(a - amax)
    denom = _mm(segT, _mm(seg, ex))
    attn = ex / denom

    gv34 = _mm(hv, bv34_ref[...]) + bv34b_ref[...]
    gvi = jnp.concatenate(
        [_mm(hv[t * N:(t + 1) * N], av0_ref[0, t]) for t in range(TI)], axis=0)
    val0 = (gvi + avb0x + s1a * gv34[:, :32] + s1b * gv34[:, 32:]) * (
        1.0 / SQRT26)

    e1 = jnp.concatenate(
        [_mm(hv[t * N:(t + 1) * N], ave_ref[0, t]) for t in range(TI)], axis=0)
    e1 = e1 + avbex
    gu = _mm(hv, u34_ref[...]) + u34b_ref[...]
    coefc = e1 / d - np.float32(0.5) * gu[:, :DV0]
    u4h = np.float32(0.5) * gu[:, DV0:]
    val1 = (_mm(coefc, r8) * _mm(ce, t3)
            - _mm(u4h, r8) * _mm(ve, t3)) * (1.0 / SQRT78)

    at32 = _mm(attn, rep8)
    at24 = _mm(attn, rep6)
    x0 = _mm(seg, at32 * val0)
    x1 = _mm(seg, at24 * val1)

    out0 = _mm(x0, wout0_ref[...]) * np.float32(1.0 / np.sqrt(32.0))
    out0_ref[0] = out0.reshape(TI, 1, 24)
    out1_ref[0] = _mm(x1, w1o_ref[...]).reshape(TI, 1, 6)

    dmat = _mmx(segT, _mmx(seg, d * eyeT))
    less = ((dmat < d) | ((dmat == d) & (io1 < jmod))).astype(f32)
    rank = jnp.sum(less, axis=1, keepdims=True)
    rank_rows = _mm(seg, rank * eyeT)
    r_io = _ii((N, N), 0).astype(f32)
    for t in range(TI):
        perm = (rank_rows[t:t + 1, :] == r_io).astype(f32)
        attn_ref[0, t] = _mm(perm, attn[t * N:(t + 1) * N])


@jax.jit
def kernel(sp, coord, veloc, t, Wq, mkW1, mkb1, mkW2, mkb2,
           mvW1, mvb1, mvW2, mvb2, w_dot, Wout0e, Wout1o):
    f32 = jnp.float32

    wk_atom = mkW2[:, :768].reshape(64, 24, 32).transpose(1, 0, 2).reshape(24, 2048)
    bk_atom = mkb2[:768].reshape(24, 32)
    bk34 = mkW2[:, 768:832]
    bk34b = mkb2[768:832].reshape(1, 64)
    wv_atom0 = mvW2[:, :768].reshape(64, 24, 32).transpose(1, 0, 2).reshape(24, 2048)
    bv_atom0 = mvb2[:768].reshape(24, 32)
    bv34 = mvW2[:, 768:832]
    bv34b = mvb2[768:832].reshape(1, 64)
    wv_atome = mvW2[:, 832:1024].reshape(64, 24, 8).transpose(1, 0, 2).reshape(24, 512)
    bv_atome = mvb2[832:1024].reshape(24, 8)
    u34 = mvW2[:, 1024:1040]
    u34b = mvb2[1024:1040].reshape(1, 16)
    wq_fold = (Wq.reshape(24, H, DK) @ w_dot).reshape(24, 32) * f32(
        1.0 / np.sqrt(24.0))
    w1o_exp = jnp.einsum('ko,xy->kxoy', Wout1o,
                         jnp.eye(3, dtype=f32)).reshape(24, 6) * f32(
                             1.0 / np.sqrt(8.0))
    zpad = jnp.zeros((SP, 64), f32)
    tk_w = jnp.concatenate([zpad, mkW1[SP:, :]], 0)
    tv_w = jnp.concatenate([zpad, mvW1[SP:, :]], 0)
    wcat = jnp.concatenate([
        wk_atom, bk_atom, wv_atom0, bv_atom0, wv_atome, bv_atome,
        wq_fold, tk_w, tv_w], axis=1)

    c24 = jnp.concatenate([sp, t], -1).reshape(B * N, 24)

    pa = pl.pallas_call(
        _peratom_kernel,
        out_shape=jax.ShapeDtypeStruct((B * N, 4840), f32),
    )(c24, wcat)

    pa = pa.reshape(B, N, 4840)
    a_k = pa[..., 0:2048].reshape(B, N, 64, 32)
    akb = pa[..., 2048:2080].reshape(B, N, 1, 32)
    a_v0 = pa[..., 2080:4128].reshape(B, N, 64, 32)
    avb0 = pa[..., 4128:4160].reshape(B, N, 1, 32)
    a_ve = pa[..., 4160:4672].reshape(B, N, 64, 8)
    avbe = pa[..., 4672:4680].reshape(B, N, 1, 8)
    qw = pa[..., 4680:4712]
    tk = pa[..., 4712:4776].reshape(B, N, 1, 64)
    tv = pa[..., 4776:4840].reshape(B, N, 1, 64)

    full2 = lambda arr: pl.BlockSpec(arr.shape, lambda b, i: (0, 0))
    perb = lambda shp: pl.BlockSpec((1,) + shp[1:], lambda b, i: (b, 0, 0))
    peri = lambda shp: pl.BlockSpec((1, TI) + shp[2:],
                                    lambda b, i: (b, i, 0, 0))

    mkb1r = mkb1.reshape(1, 64)
    mvb1r = mvb1.reshape(1, 64)

    out0, out1, attn_s = pl.pallas_call(
        _edge_kernel,
        grid=(B, N // TI),
        in_specs=[
            perb(coord.shape), perb(veloc.shape), perb(qw.shape),
            peri(a_k.shape), peri(akb.shape), peri(a_v0.shape),
            peri(avb0.shape), peri(a_ve.shape), peri(avbe.shape),
            peri(tk.shape), peri(tv.shape),
            full2(mkW1[:SP]), full2(mkb1r), full2(mvW1[:SP]), full2(mvb1r),
            full2(bk34), full2(bk34b), full2(bv34), full2(bv34b),
            full2(u34), full2(u34b), full2(Wout0e), full2(w1o_exp),
        ],
        out_specs=[
            pl.BlockSpec((1, TI, 1, 24), lambda b, i: (b, i, 0, 0)),
            pl.BlockSpec((1, TI, 1, 6), lambda b, i: (b, i, 0, 0)),
            pl.BlockSpec((1, TI, N, H), lambda b, i: (b, i, 0, 0)),
        ],
        out_shape=[
            jax.ShapeDtypeStruct((B, N, 1, 24), f32),
            jax.ShapeDtypeStruct((B, N, 1, 6), f32),
            jax.ShapeDtypeStruct((B, N, N, H), f32),
        ],
    )(coord, veloc, qw, a_k, akb, a_v0, avb0, a_ve, avbe, tk, tv,
      mkW1[:SP], mkb1r, mvW1[:SP], mvb1r, bk34, bk34b, bv34, bv34b,
      u34, u34b, Wout0e, w1o_exp)

    out0 = out0.reshape(B, N, 24)
    out1 = out1.reshape(B, N, 6)
    sp_o = out0[..., :SP]
    t_o = out0[..., SP:]
    coord_o = out1[..., :3]
    veloc_o = out1[..., 3:]
    attn = attn_s.transpose(0, 3, 1, 2)[..., None]
    return (sp_o, coord_o, veloc_o, t_o, attn)


# speedup vs baseline: 3.1951x; 1.4083x over previous
import jax
import jax.numpy as jnp
import numpy as np
from jax.experimental import pallas as pl

B, N, H = 2, 128, 4
SP, TD, DE = 16, 8, 16
CUT = 5.0
DK = 8
DV0, DV1 = 8, 2

TI = 16
S = TI * N

RSTEP = np.float32(CUT / (DE - 1))
SQRT26 = np.float32(np.sqrt(26.0))
SQRT78 = np.float32(np.sqrt(3.0) * np.sqrt(26.0))
ASCL = np.float32(1.0 / (np.sqrt(64.0) * np.sqrt(float(DK))))


def _peratom_kernel(c24_ref, wcat_ref, out_ref):
    out_ref[...] = jnp.dot(c24_ref[...], wcat_ref[...],
                           preferred_element_type=jnp.float32)


def _ii(shape, dim):
    return jax.lax.broadcasted_iota(jnp.int32, shape, dim)


def _fdiv(x, n):
    return (x.astype(jnp.float32) * np.float32(1.0 / n)).astype(jnp.int32)


def _mm(a, b):
    return jnp.dot(a, b, preferred_element_type=jnp.float32)


def _mmx(a, b):
    return jnp.dot(a, b, preferred_element_type=jnp.float32,
                   precision=jax.lax.Precision.HIGHEST)


def _tile(x):
    return jnp.broadcast_to(x[None], (TI,) + x.shape).reshape(S, x.shape[-1])


def _rep(x):
    return jnp.broadcast_to(x[:, None, :],
                            (TI, N, x.shape[-1])).reshape(S, x.shape[-1])


def _edge_kernel(coord_ref, veloc_ref, qw_ref,
                 ak_ref, akb_ref, av_ref, avb_ref,
                 tk_ref, tv_ref,
                 mk1_ref, mkb1_ref, mv1_ref, mvb1_ref,
                 bk34_ref, bk34b_ref, bv34_ref, bv34b_ref,
                 u34_ref, u34b_ref, wout0_ref, w1o_ref,
                 out0_ref, out1_ref, attn_ref):
    g = pl.program_id(1)
    f32 = jnp.float32

    io0 = _ii((S, N), 0)
    io1 = _ii((S, N), 1)
    jmod = io0 - _fdiv(io0, N) * N
    eyeT = (jmod == io1).astype(f32)
    seg = (_fdiv(_ii((TI, S), 1), N) == _ii((TI, S), 0)).astype(f32)
    selm = (_fdiv(_ii((32, H), 0), DK) == _ii((32, H), 1)).astype(f32)
    rep8 = (_fdiv(_ii((H, 32), 1), DK) == _ii((H, 32), 0)).astype(f32)
    rep6 = (_fdiv(_ii((H, 24), 1), 6) == _ii((H, 24), 0)).astype(f32)
    r8 = (_fdiv(_ii((DV0, 24), 1), 3) == _ii((DV0, 24), 0)).astype(f32)
    x24 = _ii((3, 24), 1)
    t3 = ((x24 - _fdiv(x24, 3) * 3) == _ii((3, 24), 0)).astype(f32)

    cb = coord_ref[0]
    vb = veloc_ref[0]
    cis = coord_ref[0, pl.ds(g * TI, TI), :]
    vis = veloc_ref[0, pl.ds(g * TI, TI), :]

    ce = _tile(cb) - _rep(cis)
    ve = _tile(vb) - _rep(vis)
    qw_t = _tile(qw_ref[0])

    sumsq = jnp.sum(ce * ce, axis=1, keepdims=True)
    d = jnp.sqrt(sumsq + np.float32(1e-12))
    dot_ve = jnp.sum(ve * ce, axis=1, keepdims=True)
    inv2s3d = np.float32(-0.5 / np.sqrt(3.0)) / d
    s1a = sumsq * inv2s3d
    s1b = dot_ve * inv2s3d

    kvals = _ii((1, DE), 1).astype(f32) * RSTEP
    diff = (d - kvals) * np.float32(1.0 / RSTEP)
    rbf = jnp.exp(-diff * diff) * np.float32(1.0 / 1.12)

    tkx = _rep(tk_ref[0, :, 0, :])
    tvx = _rep(tv_ref[0, :, 0, :])
    akbx = _rep(akb_ref[0, :, 0, :])
    avbx = _rep(avb_ref[0, :, 0, :])

    hk = jax.nn.silu(_mm(rbf, mk1_ref[...]) + tkx + mkb1_ref[...])
    hv = jax.nn.silu(_mm(rbf, mv1_ref[...]) + tvx + mvb1_ref[...])

    g34 = _mm(hk, bk34_ref[...]) + bk34b_ref[...]
    gi = jnp.concatenate(
        [_mm(hk[t * N:(t + 1) * N], ak_ref[0, t]) for t in range(TI)], axis=0)
    keyf = (gi + akbx + s1a * g34[:, :32] + s1b * g34[:, 32:]) * (1.0 / SQRT26)

    a = _mm(qw_t * keyf, selm) * ASCL
    amax = jnp.max(a, axis=0, keepdims=True)
    ex = jnp.exp(a - amax)
    denom = _rep(_mm(seg, ex))
    attn = ex / denom

    gv34 = _mm(hv, bv34_ref[...]) + bv34b_ref[...]
    gcomb = jnp.concatenate(
        [_mm(hv[t * N:(t + 1) * N], av_ref[0, t]) for t in range(TI)], axis=0)
    gcomb = gcomb + avbx
    val0 = (gcomb[:, :32] + s1a * gv34[:, :32] + s1b * gv34[:, 32:]) * (
        1.0 / SQRT26)
    e1 = gcomb[:, 32:]

    gu = _mm(hv, u34_ref[...]) + u34b_ref[...]
    coefc = e1 / d - np.float32(0.5) * gu[:, :DV0]
    u4h = np.float32(0.5) * gu[:, DV0:]
    val1 = (_mm(coefc, r8) * _mm(ce, t3)
            - _mm(u4h, r8) * _mm(ve, t3)) * (1.0 / SQRT78)

    at32 = _mm(attn, rep8)
    at24 = _mm(attn, rep6)
    x0 = _mm(seg, at32 * val0)
    x1 = _mm(seg, at24 * val1)

    out0 = _mm(x0, wout0_ref[...]) * np.float32(1.0 / np.sqrt(32.0))
    out0_ref[0] = out0.reshape(TI, 1, 24)
    out1_ref[0] = _mm(x1, w1o_ref[...]).reshape(TI, 1, 6)

    dmat = _rep(_mmx(seg, d * eyeT))
    less = ((dmat < d) | ((dmat == d) & (io1 < jmod))).astype(f32)
    rank = jnp.sum(less, axis=1, keepdims=True)
    ponehot = (rank == io1.astype(f32)).astype(f32)
    srt = [_mm(seg, ponehot * attn[:, h:h + 1]) for h in range(H)]
    attn_ref[0] = jnp.stack(srt, axis=-1)


@jax.jit
def kernel(sp, coord, veloc, t, Wq, mkW1, mkb1, mkW2, mkb2,
           mvW1, mvb1, mvW2, mvb2, w_dot, Wout0e, Wout1o):
    f32 = jnp.float32

    wk_atom = mkW2[:, :768].reshape(64, 24, 32).transpose(1, 0, 2).reshape(24, 2048)
    bk_atom = mkb2[:768].reshape(24, 32)
    bk34 = mkW2[:, 768:832]
    bk34b = mkb2[768:832].reshape(1, 64)
    wv_comb = jnp.concatenate(
        [mvW2[:, :768].reshape(64, 24, 32),
         mvW2[:, 832:1024].reshape(64, 24, 8)],
        axis=2).transpose(1, 0, 2).reshape(24, 2560)
    bv_comb = jnp.concatenate(
        [mvb2[:768].reshape(24, 32), mvb2[832:1024].reshape(24, 8)], axis=1)
    bv34 = mvW2[:, 768:832]
    bv34b = mvb2[768:832].reshape(1, 64)
    u34 = mvW2[:, 1024:1040]
    u34b = mvb2[1024:1040].reshape(1, 16)
    wq_fold = (Wq.reshape(24, H, DK) @ w_dot).reshape(24, 32) * f32(
        1.0 / np.sqrt(24.0))
    w1o_exp = jnp.einsum('ko,xy->kxoy', Wout1o,
                         jnp.eye(3, dtype=f32)).reshape(24, 6) * f32(
                             1.0 / np.sqrt(8.0))
    zpad = jnp.zeros((SP, 64), f32)
    tk_w = jnp.concatenate([zpad, mkW1[SP:, :]], 0)
    tv_w = jnp.concatenate([zpad, mvW1[SP:, :]], 0)
    wcat = jnp.concatenate([
        wk_atom, bk_atom, wv_comb, bv_comb,
        wq_fold, tk_w, tv_w], axis=1)

    c24 = jnp.concatenate([sp, t], -1).reshape(B * N, 24)

    pa = pl.pallas_call(
        _peratom_kernel,
        out_shape=jax.ShapeDtypeStruct((B * N, 4840), f32),
    )(c24, wcat)

    pa = pa.reshape(B, N, 4840)
    a_k = pa[..., 0:2048].reshape(B, N, 64, 32)
    akb = pa[..., 2048:2080].reshape(B, N, 1, 32)
    a_v = pa[..., 2080:4640].reshape(B, N, 64, 40)
    avb = pa[..., 4640:4680].reshape(B, N, 1, 40)
    qw = pa[..., 4680:4712]
    tk = pa[..., 4712:4776].reshape(B, N, 1, 64)
    tv = pa[..., 4776:4840].reshape(B, N, 1, 64)

    full2 = lambda arr: pl.BlockSpec(arr.shape, lambda b, i: (0, 0))
    perb = lambda shp: pl.BlockSpec((1,) + shp[1:], lambda b, i: (b, 0, 0))
    peri = lambda shp: pl.BlockSpec((1, TI) + shp[2:],
                                    lambda b, i: (b, i, 0, 0))

    mkb1r = mkb1.reshape(1, 64)
    mvb1r = mvb1.reshape(1, 64)

    out0, out1, attn_s = pl.pallas_call(
        _edge_kernel,
        grid=(B, N // TI),
        in_specs=[
            perb(coord.shape), perb(veloc.shape), perb(qw.shape),
            peri(a_k.shape), peri(akb.shape), peri(a_v.shape),
            peri(avb.shape),
            peri(tk.shape), peri(tv.shape),
            full2(mkW1[:SP]), full2(mkb1r), full2(mvW1[:SP]), full2(mvb1r),
            full2(bk34), full2(bk34b), full2(bv34), full2(bv34b),
            full2(u34), full2(u34b), full2(Wout0e), full2(w1o_exp),
        ],
        out_specs=[
            pl.BlockSpec((1, TI, 1, 24), lambda b, i: (b, i, 0, 0)),
            pl.BlockSpec((1, TI, 1, 6), lambda b, i: (b, i, 0, 0)),
            pl.BlockSpec((1, TI, N, H), lambda b, i: (b, i, 0, 0)),
        ],
        out_shape=[
            jax.ShapeDtypeStruct((B, N, 1, 24), f32),
            jax.ShapeDtypeStruct((B, N, 1, 6), f32),
            jax.ShapeDtypeStruct((B, N, N, H), f32),
        ],
    )(coord, veloc, qw, a_k, akb, a_v, avb, tk, tv,
      mkW1[:SP], mkb1r, mvW1[:SP], mvb1r, bk34, bk34b, bv34, bv34b,
      u34, u34b, Wout0e, w1o_exp)

    out0 = out0.reshape(B, N, 24)
    out1 = out1.reshape(B, N, 6)
    sp_o = out0[..., :SP]
    t_o = out0[..., SP:]
    coord_o = out1[..., :3]
    veloc_o = out1[..., 3:]
    attn = attn_s.transpose(0, 3, 1, 2)[..., None]
    return (sp_o, coord_o, veloc_o, t_o, attn)
